# baseline (speedup 1.0000x reference)
import jax
import jax.numpy as jnp
from jax import lax
from jax.experimental import pallas as pl
from jax.experimental.pallas import tpu as pltpu
from jax.experimental.pallas import tpu_sc as plsc

_SCORE_THRESH = 0.05
_NMS_THRESH = 0.5
_MAX_DET = 100

_N = 5000
_NPAD = 5008
_NG = _NPAD // 16
_KCAP = 112
_REJ = 6 * _KCAP
_BUF = _REJ + 5 * 128
_OUT_FLAT = 512

_GDN = lax.GatherDimensionNumbers(
    offset_dims=(), collapsed_slice_dims=(0,), start_index_map=(0,))


def _dyn_bcast(vec, ivec):
    return lax.gather(vec, ivec[:, None], _GDN, slice_sizes=(1,),
                      mode=lax.GatherScatterMode.PROMISE_IN_BOUNDS)


def _nms_body(bh, sh, oh, outh, bv, sv, ov, kbuf, obuf):
    wid = lax.axis_index("s") + lax.axis_index("c")

    @pl.when(wid == 0)
    def _():
        pltpu.sync_copy(bh, bv)
        pltpu.sync_copy(sh, sv)
        pltpu.sync_copy(oh, ov)

    iota = lax.broadcasted_iota(jnp.int32, (16,), 0)
    lane_lt5 = iota < 5
    lane_lt6 = iota < 6
    zero16 = jnp.zeros((16,), jnp.float32)
    for q in range(5 * _KCAP // 16):
        kbuf[pl.ds(q * 16, 16)] = zero16

    def group_step(g, st):
        run = jnp.logical_and(st[0] < _MAX_DET, wid == 0)
        return lax.cond(run, lambda: group_body(g, st), lambda: st)

    def group_body(g, st):
        nk, nr = st
        idxv = ov[pl.ds(g * 16, 16)]
        bidx = jnp.minimum(idxv, _N - 1) * 4
        gx1 = plsc.load_gather(bv, [bidx])
        gy1 = plsc.load_gather(bv, [bidx + 1])
        gx2 = plsc.load_gather(bv, [bidx + 2])
        gy2 = plsc.load_gather(bv, [bidx + 3])
        gs = plsc.load_gather(sv, [idxv])

        def lane_body(i, st2):
            nk2, nr2 = st2
            ivec = jnp.full((16,), i, jnp.int32)
            cx1v = _dyn_bcast(gx1, ivec)
            cy1v = _dyn_bcast(gy1, ivec)
            cx2v = _dyn_bcast(gx2, ivec)
            cy2v = _dyn_bcast(gy2, ivec)
            csv = _dyn_bcast(gs, ivec)
            valid = jnp.any(csv > _SCORE_THRESH)
            cav = (cx2v - cx1v) * (cy2v - cy1v)

            def iou_vreg(j, acc):
                kx1j = kbuf[pl.ds(j * 16, 16)]
                ky1j = kbuf[pl.ds(_KCAP + j * 16, 16)]
                kx2j = kbuf[pl.ds(2 * _KCAP + j * 16, 16)]
                ky2j = kbuf[pl.ds(3 * _KCAP + j * 16, 16)]
                karj = kbuf[pl.ds(4 * _KCAP + j * 16, 16)]
                ltx = jnp.maximum(kx1j, cx1v)
                lty = jnp.maximum(ky1j, cy1v)
                rbx = jnp.minimum(kx2j, cx2v)
                rby = jnp.minimum(ky2j, cy2v)
                w = jnp.maximum(rbx - ltx, 0.0)
                h = jnp.maximum(rby - lty, 0.0)
                inter = w * h
                union = (karj + cav) - inter
                iou = inter / jnp.maximum(union, 1e-9)
                return jnp.logical_or(acc, iou > _NMS_THRESH)

            acc = iou_vreg(0, jnp.zeros((16,), jnp.bool_))
            for j in range(1, _KCAP // 16):
                acc = lax.cond(j * 16 < nk2,
                               lambda a, jj=j: iou_vreg(jj, a),
                               lambda a: a, acc)
            keep = jnp.logical_and(valid, jnp.logical_not(jnp.any(acc)))
            keepv = jnp.broadcast_to(keep, (16,))

            v01 = jnp.where(iota == 0, cx1v, cy1v)
            v012 = jnp.where(iota <= 1, v01, cx2v)
            coords = jnp.where(iota <= 2, v012, cy2v)

            kvec = jnp.where(iota <= 3, coords,
                             jnp.where(iota == 4, cav, csv))
            kslot = jnp.minimum(nk2, _KCAP - 1)
            plsc.store_scatter(kbuf, [iota * _KCAP + kslot], kvec,
                               mask=jnp.logical_and(lane_lt6, keepv))

            rvec = jnp.where(iota <= 3, coords, 0.0)
            rslot = jnp.minimum(nr2, _MAX_DET)
            plsc.store_scatter(kbuf, [_REJ + iota * 128 + rslot], rvec,
                               mask=jnp.logical_and(lane_lt5,
                                                    jnp.logical_not(keepv)))
            ki = keep.astype(jnp.int32)
            return nk2 + ki, nr2 + (1 - ki)

        st = (nk, nr)
        for i in range(16):
            st = lane_body(i, st)
        return st

    nk, _ = lax.fori_loop(0, _NG, group_step, (jnp.int32(0), jnp.int32(0)))

    @pl.when(wid == 0)
    def _():
        for t in range(_OUT_FLAT // 16):
            flat = t * 16 + iota
            p = flat // 5
            c = flat - p * 5
            ck = jnp.where(c == 4, 5, c)
            src_k = ck * _KCAP + p
            src_r = _REJ + c * 128 + jnp.maximum(p - nk, 0)
            src = jnp.where(p < nk, src_k, src_r)
            obuf[pl.ds(t * 16, 16)] = plsc.load_gather(kbuf, [src])
        pltpu.sync_copy(obuf, outh)


@jax.jit
def kernel(boxes, scores):
    s = jnp.where(scores > _SCORE_THRESH, scores, -1.0)
    order = jnp.argsort(-s).astype(jnp.int32)
    pad_i = jnp.arange(_N, _NPAD, dtype=jnp.int32)
    order_p = jnp.concatenate([order, pad_i])
    s_p = jnp.concatenate([s, jnp.full((_NPAD - _N,), -1.0, jnp.float32)])
    bflat = boxes.reshape(-1)

    mesh = plsc.VectorSubcoreMesh(core_axis_name="c", subcore_axis_name="s",
                                  num_cores=1)
    f = pl.kernel(
        _nms_body,
        out_type=jax.ShapeDtypeStruct((_OUT_FLAT,), jnp.float32),
        mesh=mesh,
        compiler_params=pltpu.CompilerParams(needs_layout_passes=False),
        scratch_types=[
            pltpu.VMEM((4 * _N,), jnp.float32),
            pltpu.VMEM((_NPAD,), jnp.float32),
            pltpu.VMEM((_NPAD,), jnp.int32),
            pltpu.VMEM((_BUF,), jnp.float32),
            pltpu.VMEM((_OUT_FLAT,), jnp.float32),
        ],
    )
    out_flat = f(bflat, s_p, order_p)
    return out_flat[:_MAX_DET * 5].reshape(_MAX_DET, 5)


# speedup vs baseline: 1.0814x; 1.0814x over previous
import jax
import jax.numpy as jnp
from jax import lax
from jax.experimental import pallas as pl
from jax.experimental.pallas import tpu as pltpu
from jax.experimental.pallas import tpu_sc as plsc

_SCORE_THRESH = 0.05
_NMS_THRESH = 0.5
_MAX_DET = 100

_N = 5000
_NPAD = 5008
_NG = _NPAD // 16
_KCAP = 112
_REJ = 6 * _KCAP
_BUF = _REJ + 5 * 128
_OUT_FLAT = 512

_GDN = lax.GatherDimensionNumbers(
    offset_dims=(), collapsed_slice_dims=(0,), start_index_map=(0,))


def _dyn_bcast(vec, ivec):
    return lax.gather(vec, ivec[:, None], _GDN, slice_sizes=(1,),
                      mode=lax.GatherScatterMode.PROMISE_IN_BOUNDS)


def _nms_body(bh, sh, oh, outh, bv, sv, ov, kbuf, obuf):
    wid = lax.axis_index("s") + lax.axis_index("c")

    @pl.when(wid == 0)
    def _():
        pltpu.sync_copy(bh, bv)
        pltpu.sync_copy(sh, sv)
        pltpu.sync_copy(oh, ov)

    iota = lax.broadcasted_iota(jnp.int32, (16,), 0)
    lane_lt5 = iota < 5
    lane_lt6 = iota < 6
    zero16 = jnp.zeros((16,), jnp.float32)
    for q in range(5 * _KCAP // 16):
        kbuf[pl.ds(q * 16, 16)] = zero16

    def group_step(g, st):
        run = jnp.logical_and(st[0] < _MAX_DET, wid == 0)
        return lax.cond(run, lambda: group_body(g, st), lambda: st)

    def group_body(g, st):
        nk, nr = st
        idxv = ov[pl.ds(g * 16, 16)]
        bidx = jnp.minimum(idxv, _N - 1) * 4
        gx1 = plsc.load_gather(bv, [bidx])
        gy1 = plsc.load_gather(bv, [bidx + 1])
        gx2 = plsc.load_gather(bv, [bidx + 2])
        gy2 = plsc.load_gather(bv, [bidx + 3])
        gs = plsc.load_gather(sv, [idxv])

        def lane_body(i, st2):
            nk2, nr2 = st2
            ivec = jnp.broadcast_to(i, (16,))
            cx1v = _dyn_bcast(gx1, ivec)
            cy1v = _dyn_bcast(gy1, ivec)
            cx2v = _dyn_bcast(gx2, ivec)
            cy2v = _dyn_bcast(gy2, ivec)
            csv = _dyn_bcast(gs, ivec)
            valid = jnp.any(csv > _SCORE_THRESH)
            cav = (cx2v - cx1v) * (cy2v - cy1v)

            def iou_vreg(j, acc):
                kx1j = kbuf[pl.ds(j * 16, 16)]
                ky1j = kbuf[pl.ds(_KCAP + j * 16, 16)]
                kx2j = kbuf[pl.ds(2 * _KCAP + j * 16, 16)]
                ky2j = kbuf[pl.ds(3 * _KCAP + j * 16, 16)]
                karj = kbuf[pl.ds(4 * _KCAP + j * 16, 16)]
                ltx = jnp.maximum(kx1j, cx1v)
                lty = jnp.maximum(ky1j, cy1v)
                rbx = jnp.minimum(kx2j, cx2v)
                rby = jnp.minimum(ky2j, cy2v)
                w = jnp.maximum(rbx - ltx, 0.0)
                h = jnp.maximum(rby - lty, 0.0)
                inter = w * h
                union = (karj + cav) - inter
                iou = inter / jnp.maximum(union, 1e-9)
                return jnp.logical_or(acc, iou > _NMS_THRESH)

            acc = iou_vreg(0, jnp.zeros((16,), jnp.bool_))
            for j in range(1, _KCAP // 16):
                acc = lax.cond(j * 16 < nk2,
                               lambda a, jj=j: iou_vreg(jj, a),
                               lambda a: a, acc)
            keep = jnp.logical_and(valid, jnp.logical_not(jnp.any(acc)))
            keepv = jnp.broadcast_to(keep, (16,))

            v01 = jnp.where(iota == 0, cx1v, cy1v)
            v012 = jnp.where(iota <= 1, v01, cx2v)
            coords = jnp.where(iota <= 2, v012, cy2v)

            kvec = jnp.where(iota <= 3, coords,
                             jnp.where(iota == 4, cav, csv))
            kslot = jnp.minimum(nk2, _KCAP - 1)
            plsc.store_scatter(kbuf, [iota * _KCAP + kslot], kvec,
                               mask=jnp.logical_and(lane_lt6, keepv))

            rvec = jnp.where(iota <= 3, coords, 0.0)
            rslot = jnp.minimum(nr2, _MAX_DET)
            plsc.store_scatter(kbuf, [_REJ + iota * 128 + rslot], rvec,
                               mask=jnp.logical_and(lane_lt5,
                                                    jnp.logical_not(keepv)))
            ki = keep.astype(jnp.int32)
            return nk2 + ki, nr2 + (1 - ki)

        return lax.fori_loop(0, 16, lane_body, (nk, nr))

    nk, _ = lax.fori_loop(0, _NG, group_step, (jnp.int32(0), jnp.int32(0)))

    @pl.when(wid == 0)
    def _():
        for t in range(_OUT_FLAT // 16):
            flat = t * 16 + iota
            p = flat // 5
            c = flat - p * 5
            ck = jnp.where(c == 4, 5, c)
            src_k = ck * _KCAP + p
            src_r = _REJ + c * 128 + jnp.maximum(p - nk, 0)
            src = jnp.where(p < nk, src_k, src_r)
            obuf[pl.ds(t * 16, 16)] = plsc.load_gather(kbuf, [src])
        pltpu.sync_copy(obuf, outh)


@jax.jit
def kernel(boxes, scores):
    s = jnp.where(scores > _SCORE_THRESH, scores, -1.0)
    order = jnp.argsort(-s).astype(jnp.int32)
    pad_i = jnp.arange(_N, _NPAD, dtype=jnp.int32)
    order_p = jnp.concatenate([order, pad_i])
    s_p = jnp.concatenate([s, jnp.full((_NPAD - _N,), -1.0, jnp.float32)])
    bflat = boxes.reshape(-1)

    mesh = plsc.VectorSubcoreMesh(core_axis_name="c", subcore_axis_name="s",
                                  num_cores=1)
    f = pl.kernel(
        _nms_body,
        out_type=jax.ShapeDtypeStruct((_OUT_FLAT,), jnp.float32),
        mesh=mesh,
        compiler_params=pltpu.CompilerParams(needs_layout_passes=False),
        scratch_types=[
            pltpu.VMEM((4 * _N,), jnp.float32),
            pltpu.VMEM((_NPAD,), jnp.float32),
            pltpu.VMEM((_NPAD,), jnp.int32),
            pltpu.VMEM((_BUF,), jnp.float32),
            pltpu.VMEM((_OUT_FLAT,), jnp.float32),
        ],
    )
    out_flat = f(bflat, s_p, order_p)
    return out_flat[:_MAX_DET * 5].reshape(_MAX_DET, 5)


# speedup vs baseline: 1.1471x; 1.0607x over previous
import jax
import jax.numpy as jnp
from jax import lax
from jax.experimental import pallas as pl
from jax.experimental.pallas import tpu as pltpu
from jax.experimental.pallas import tpu_sc as plsc

_SCORE_THRESH = 0.05
_NMS_THRESH = 0.5
_MAX_DET = 100

_N = 5000
_NPAD = 5120
_NG = _NPAD // 16
_NB = _NG // 16
_KCAP = 112
_REJ = 6 * _KCAP
_BUF = _REJ + 5 * 128
_OUT_FLAT = 512

_GDN = lax.GatherDimensionNumbers(
    offset_dims=(), collapsed_slice_dims=(0,), start_index_map=(0,))


def _dyn_bcast(vec, ivec):
    return lax.gather(vec, ivec[:, None], _GDN, slice_sizes=(1,),
                      mode=lax.GatherScatterMode.PROMISE_IN_BOUNDS)


def _nms_body(bh, sh, oh, outh, bv, sv, ov, kbuf, obuf):
    wid = lax.axis_index("s") + lax.axis_index("c")

    @pl.when(wid == 0)
    def _():
        pltpu.sync_copy(bh, bv)
        pltpu.sync_copy(sh, sv)
        pltpu.sync_copy(oh, ov)

    iota = lax.broadcasted_iota(jnp.int32, (16,), 0)
    lane_lt5 = iota < 5
    lane_lt6 = iota < 6
    zero16 = jnp.zeros((16,), jnp.float32)
    for q in range(5 * _KCAP // 16):
        kbuf[pl.ds(q * 16, 16)] = zero16

    def block_step(b, st):
        run = jnp.logical_and(st[0] < _MAX_DET, wid == 0)
        return lax.cond(run, lambda: lax.fori_loop(
            b * 16, b * 16 + 16, group_step, st), lambda: st)

    def group_step(g, st):
        return lax.cond(st[0] < _MAX_DET, lambda: group_body(g, st),
                        lambda: st)

    def group_body(g, st):
        nk, nr = st
        idxv = ov[pl.ds(g * 16, 16)]
        bidx = jnp.minimum(idxv, _N - 1) * 4
        gx1 = plsc.load_gather(bv, [bidx])
        gy1 = plsc.load_gather(bv, [bidx + 1])
        gx2 = plsc.load_gather(bv, [bidx + 2])
        gy2 = plsc.load_gather(bv, [bidx + 3])
        gs = plsc.load_gather(sv, [idxv])

        def lane_body(i, st2):
            nk2, nr2 = st2
            ivec = jnp.broadcast_to(i, (16,))
            cx1v = _dyn_bcast(gx1, ivec)
            cy1v = _dyn_bcast(gy1, ivec)
            cx2v = _dyn_bcast(gx2, ivec)
            cy2v = _dyn_bcast(gy2, ivec)
            csv = _dyn_bcast(gs, ivec)
            cav = (cx2v - cx1v) * (cy2v - cy1v)

            def iou_vreg(j, acc):
                kx1j = kbuf[pl.ds(j * 16, 16)]
                ky1j = kbuf[pl.ds(_KCAP + j * 16, 16)]
                kx2j = kbuf[pl.ds(2 * _KCAP + j * 16, 16)]
                ky2j = kbuf[pl.ds(3 * _KCAP + j * 16, 16)]
                karj = kbuf[pl.ds(4 * _KCAP + j * 16, 16)]
                ltx = jnp.maximum(kx1j, cx1v)
                lty = jnp.maximum(ky1j, cy1v)
                rbx = jnp.minimum(kx2j, cx2v)
                rby = jnp.minimum(ky2j, cy2v)
                w = jnp.maximum(rbx - ltx, 0.0)
                h = jnp.maximum(rby - lty, 0.0)
                inter = w * h
                union = (karj + cav) - inter
                iou = inter / jnp.maximum(union, 1e-9)
                return jnp.logical_or(acc, iou > _NMS_THRESH)

            acc = iou_vreg(0, csv <= _SCORE_THRESH)
            for j in range(1, _KCAP // 16):
                acc = lax.cond(j * 16 < nk2,
                               lambda a, jj=j: iou_vreg(jj, a),
                               lambda a: a, acc)
            keep = jnp.logical_not(jnp.any(acc))
            keepv = jnp.broadcast_to(keep, (16,))

            v01 = jnp.where(iota == 0, cx1v, cy1v)
            v012 = jnp.where(iota <= 1, v01, cx2v)
            coords = jnp.where(iota <= 2, v012, cy2v)

            kvec = jnp.where(iota <= 3, coords,
                             jnp.where(iota == 4, cav, csv))
            kslot = jnp.minimum(nk2, _KCAP - 1)
            plsc.store_scatter(kbuf, [iota * _KCAP + kslot], kvec,
                               mask=jnp.logical_and(lane_lt6, keepv))

            rvec = jnp.where(iota <= 3, coords, 0.0)
            rslot = jnp.minimum(nr2, _MAX_DET)
            plsc.store_scatter(kbuf, [_REJ + iota * 128 + rslot], rvec,
                               mask=jnp.logical_and(lane_lt5,
                                                    jnp.logical_not(keepv)))
            ki = keep.astype(jnp.int32)
            return nk2 + ki, nr2 + (1 - ki)

        return lax.fori_loop(0, 16, lane_body, (nk, nr))

    nk, _ = lax.fori_loop(0, _NB, block_step, (jnp.int32(0), jnp.int32(0)))

    @pl.when(wid == 0)
    def _():
        for t in range(_OUT_FLAT // 16):
            flat = t * 16 + iota
            p = flat // 5
            c = flat - p * 5
            ck = jnp.where(c == 4, 5, c)
            src_k = ck * _KCAP + p
            src_r = _REJ + c * 128 + jnp.maximum(p - nk, 0)
            src = jnp.where(p < nk, src_k, src_r)
            obuf[pl.ds(t * 16, 16)] = plsc.load_gather(kbuf, [src])
        pltpu.sync_copy(obuf, outh)


@jax.jit
def kernel(boxes, scores):
    s = jnp.where(scores > _SCORE_THRESH, scores, -1.0)
    order = jnp.argsort(-s).astype(jnp.int32)
    pad_i = jnp.arange(_N, _NPAD, dtype=jnp.int32)
    order_p = jnp.concatenate([order, pad_i])
    s_p = jnp.concatenate([s, jnp.full((_NPAD - _N,), -1.0, jnp.float32)])
    bflat = boxes.reshape(-1)

    mesh = plsc.VectorSubcoreMesh(core_axis_name="c", subcore_axis_name="s",
                                  num_cores=1)
    f = pl.kernel(
        _nms_body,
        out_type=jax.ShapeDtypeStruct((_OUT_FLAT,), jnp.float32),
        mesh=mesh,
        compiler_params=pltpu.CompilerParams(needs_layout_passes=False),
        scratch_types=[
            pltpu.VMEM((4 * _N,), jnp.float32),
            pltpu.VMEM((_NPAD,), jnp.float32),
            pltpu.VMEM((_NPAD,), jnp.int32),
            pltpu.VMEM((_BUF,), jnp.float32),
            pltpu.VMEM((_OUT_FLAT,), jnp.float32),
        ],
    )
    out_flat = f(bflat, s_p, order_p)
    return out_flat[:_MAX_DET * 5].reshape(_MAX_DET, 5)


# speedup vs baseline: 1.1853x; 1.0333x over previous
import jax
import jax.numpy as jnp
from jax import lax
from jax.experimental import pallas as pl
from jax.experimental.pallas import tpu as pltpu
from jax.experimental.pallas import tpu_sc as plsc

_SCORE_THRESH = 0.05
_NMS_THRESH = 0.5
_MAX_DET = 100

_N = 5000
_NPAD = 5120
_NG = _NPAD // 16
_NB = _NG // 16
_KCAP = 112
_REJ = 6 * _KCAP
_BUF = _REJ + 5 * 128
_OUT_FLAT = 512

_GDN = lax.GatherDimensionNumbers(
    offset_dims=(), collapsed_slice_dims=(0,), start_index_map=(0,))


def _dyn_bcast(vec, ivec):
    return lax.gather(vec, ivec[:, None], _GDN, slice_sizes=(1,),
                      mode=lax.GatherScatterMode.PROMISE_IN_BOUNDS)


def _nms_body(bh, sh, oh, outh, bv, sv, ov, kbuf, obuf):
    wid = lax.axis_index("s") + lax.axis_index("c")

    @pl.when(wid == 0)
    def _():
        pltpu.sync_copy(bh, bv)
        pltpu.sync_copy(sh, sv.at[pl.ds(0, _N)])
        pltpu.sync_copy(oh, ov.at[pl.ds(0, _N)])

    iota = lax.broadcasted_iota(jnp.int32, (16,), 0)
    neg16 = jnp.full((16,), -1.0, jnp.float32)
    n16 = jnp.full((16,), _N, jnp.int32)
    for off in (5000, 5016, 5032, 5048, 5064, 5080, 5096, 5104):
        sv[pl.ds(off, 16)] = neg16
        ov[pl.ds(off, 16)] = n16
    lane_lt5 = iota < 5
    lane_lt6 = iota < 6
    zero16 = jnp.zeros((16,), jnp.float32)
    for q in range(5 * _KCAP // 16):
        kbuf[pl.ds(q * 16, 16)] = zero16

    def block_step(b, st):
        run = jnp.logical_and(st[0] < _MAX_DET, wid == 0)
        return lax.cond(run, lambda: lax.fori_loop(
            b * 16, b * 16 + 16, group_step, st), lambda: st)

    def group_step(g, st):
        return lax.cond(st[0] < _MAX_DET, lambda: group_body(g, st),
                        lambda: st)

    def group_body(g, st):
        nk, nr = st
        idxv = ov[pl.ds(g * 16, 16)]
        bidx = jnp.minimum(idxv, _N - 1) * 4
        gx1 = plsc.load_gather(bv, [bidx])
        gy1 = plsc.load_gather(bv, [bidx + 1])
        gx2 = plsc.load_gather(bv, [bidx + 2])
        gy2 = plsc.load_gather(bv, [bidx + 3])
        gs = plsc.load_gather(sv, [idxv])

        def lane_body(i, st2):
            nk2, nr2 = st2
            ivec = jnp.broadcast_to(i, (16,))
            cx1v = _dyn_bcast(gx1, ivec)
            cy1v = _dyn_bcast(gy1, ivec)
            cx2v = _dyn_bcast(gx2, ivec)
            cy2v = _dyn_bcast(gy2, ivec)
            csv = _dyn_bcast(gs, ivec)
            cav = (cx2v - cx1v) * (cy2v - cy1v)

            def iou_vreg(j, acc):
                kx1j = kbuf[pl.ds(j * 16, 16)]
                ky1j = kbuf[pl.ds(_KCAP + j * 16, 16)]
                kx2j = kbuf[pl.ds(2 * _KCAP + j * 16, 16)]
                ky2j = kbuf[pl.ds(3 * _KCAP + j * 16, 16)]
                karj = kbuf[pl.ds(4 * _KCAP + j * 16, 16)]
                ltx = jnp.maximum(kx1j, cx1v)
                lty = jnp.maximum(ky1j, cy1v)
                rbx = jnp.minimum(kx2j, cx2v)
                rby = jnp.minimum(ky2j, cy2v)
                w = jnp.maximum(rbx - ltx, 0.0)
                h = jnp.maximum(rby - lty, 0.0)
                inter = w * h
                union = (karj + cav) - inter
                iou = inter / jnp.maximum(union, 1e-9)
                return jnp.logical_or(acc, iou > _NMS_THRESH)

            acc = iou_vreg(0, csv <= _SCORE_THRESH)
            for j in range(1, _KCAP // 16):
                acc = lax.cond(j * 16 < nk2,
                               lambda a, jj=j: iou_vreg(jj, a),
                               lambda a: a, acc)
            keep = jnp.logical_not(jnp.any(acc))
            keepv = jnp.broadcast_to(keep, (16,))

            v01 = jnp.where(iota == 0, cx1v, cy1v)
            v012 = jnp.where(iota <= 1, v01, cx2v)
            coords = jnp.where(iota <= 2, v012, cy2v)

            kvec = jnp.where(iota <= 3, coords,
                             jnp.where(iota == 4, cav, csv))
            kslot = jnp.minimum(nk2, _KCAP - 1)
            plsc.store_scatter(kbuf, [iota * _KCAP + kslot], kvec,
                               mask=jnp.logical_and(lane_lt6, keepv))

            rvec = jnp.where(iota <= 3, coords, 0.0)
            rslot = jnp.minimum(nr2, _MAX_DET)
            plsc.store_scatter(kbuf, [_REJ + iota * 128 + rslot], rvec,
                               mask=jnp.logical_and(lane_lt5,
                                                    jnp.logical_not(keepv)))
            ki = keep.astype(jnp.int32)
            return nk2 + ki, nr2 + (1 - ki)

        return lax.fori_loop(0, 16, lane_body, (nk, nr))

    nk, _ = lax.fori_loop(0, _NB, block_step, (jnp.int32(0), jnp.int32(0)))

    @pl.when(wid == 0)
    def _():
        for t in range(_OUT_FLAT // 16):
            flat = t * 16 + iota
            p = flat // 5
            c = flat - p * 5
            ck = jnp.where(c == 4, 5, c)
            src_k = ck * _KCAP + p
            src_r = _REJ + c * 128 + jnp.maximum(p - nk, 0)
            src = jnp.where(p < nk, src_k, src_r)
            obuf[pl.ds(t * 16, 16)] = plsc.load_gather(kbuf, [src])
        pltpu.sync_copy(obuf, outh)


@jax.jit
def kernel(boxes, scores):
    key = jnp.where(scores > _SCORE_THRESH, -scores, 1.0)
    order = jnp.argsort(key).astype(jnp.int32)
    bflat = boxes.reshape(-1)

    mesh = plsc.VectorSubcoreMesh(core_axis_name="c", subcore_axis_name="s",
                                  num_cores=1)
    f = pl.kernel(
        _nms_body,
        out_type=jax.ShapeDtypeStruct((_OUT_FLAT,), jnp.float32),
        mesh=mesh,
        compiler_params=pltpu.CompilerParams(needs_layout_passes=False),
        scratch_types=[
            pltpu.VMEM((4 * _N,), jnp.float32),
            pltpu.VMEM((_NPAD,), jnp.float32),
            pltpu.VMEM((_NPAD,), jnp.int32),
            pltpu.VMEM((_BUF,), jnp.float32),
            pltpu.VMEM((_OUT_FLAT,), jnp.float32),
        ],
    )
    out_flat = f(bflat, scores, order)
    return out_flat[:_MAX_DET * 5].reshape(_MAX_DET, 5)


# speedup vs baseline: 1.2033x; 1.0152x over previous
import jax
import jax.numpy as jnp
from jax import lax
from jax.experimental import pallas as pl
from jax.experimental.pallas import tpu as pltpu
from jax.experimental.pallas import tpu_sc as plsc

_SCORE_THRESH = 0.05
_NMS_THRESH = 0.5
_MAX_DET = 100

_N = 5000
_NPAD = 5120
_NG = _NPAD // 16
_NB = _NG // 16
_KCAP = 112
_REJ = 6 * _KCAP
_BUF = _REJ + 5 * 128
_OUT_FLAT = 512

_GDN = lax.GatherDimensionNumbers(
    offset_dims=(), collapsed_slice_dims=(0,), start_index_map=(0,))


def _dyn_bcast(vec, ivec):
    return lax.gather(vec, ivec[:, None], _GDN, slice_sizes=(1,),
                      mode=lax.GatherScatterMode.PROMISE_IN_BOUNDS)


def _nms_body(bh, sh, oh, outh, bv, sv, ov, kbuf, obuf):
    wid = lax.axis_index("s") + lax.axis_index("c")

    @pl.when(wid == 0)
    def _():
        pltpu.sync_copy(bh, bv)
        pltpu.sync_copy(sh, sv.at[pl.ds(0, _N)])
        pltpu.sync_copy(oh, ov.at[pl.ds(0, _N)])

    iota = lax.broadcasted_iota(jnp.int32, (16,), 0)
    neg16 = jnp.full((16,), -1.0, jnp.float32)
    n16 = jnp.full((16,), _N, jnp.int32)

    def pad_step(k, carry):
        off = jnp.minimum(5000 + k * 16, _NPAD - 16)
        sv[pl.ds(off, 16)] = neg16
        ov[pl.ds(off, 16)] = n16
        return carry

    lax.fori_loop(0, 8, pad_step, jnp.int32(0))
    lane_lt5 = iota < 5
    lane_lt6 = iota < 6
    zero16 = jnp.zeros((16,), jnp.float32)

    def zinit_step(q, carry):
        kbuf[pl.ds(q * 16, 16)] = zero16
        return carry

    lax.fori_loop(0, 5 * _KCAP // 16, zinit_step, jnp.int32(0))

    def block_step(b, st):
        run = jnp.logical_and(st[0] < _MAX_DET, wid == 0)
        return lax.cond(run, lambda: lax.fori_loop(
            b * 16, b * 16 + 16, group_step, st), lambda: st)

    def group_step(g, st):
        return lax.cond(st[0] < _MAX_DET, lambda: group_body(g, st),
                        lambda: st)

    def group_body(g, st):
        nk, nr = st
        idxv = ov[pl.ds(g * 16, 16)]
        bidx = jnp.minimum(idxv, _N - 1) * 4
        gx1 = plsc.load_gather(bv, [bidx])
        gy1 = plsc.load_gather(bv, [bidx + 1])
        gx2 = plsc.load_gather(bv, [bidx + 2])
        gy2 = plsc.load_gather(bv, [bidx + 3])
        gs = plsc.load_gather(sv, [idxv])

        def lane_body(i, st2):
            nk2, nr2 = st2
            ivec = jnp.broadcast_to(i, (16,))
            cx1v = _dyn_bcast(gx1, ivec)
            cy1v = _dyn_bcast(gy1, ivec)
            cx2v = _dyn_bcast(gx2, ivec)
            cy2v = _dyn_bcast(gy2, ivec)
            csv = _dyn_bcast(gs, ivec)
            cav = (cx2v - cx1v) * (cy2v - cy1v)

            def iou_vreg(j, acc):
                kx1j = kbuf[pl.ds(j * 16, 16)]
                ky1j = kbuf[pl.ds(_KCAP + j * 16, 16)]
                kx2j = kbuf[pl.ds(2 * _KCAP + j * 16, 16)]
                ky2j = kbuf[pl.ds(3 * _KCAP + j * 16, 16)]
                karj = kbuf[pl.ds(4 * _KCAP + j * 16, 16)]
                ltx = jnp.maximum(kx1j, cx1v)
                lty = jnp.maximum(ky1j, cy1v)
                rbx = jnp.minimum(kx2j, cx2v)
                rby = jnp.minimum(ky2j, cy2v)
                w = jnp.maximum(rbx - ltx, 0.0)
                h = jnp.maximum(rby - lty, 0.0)
                inter = w * h
                union = (karj + cav) - inter
                iou = inter / jnp.maximum(union, 1e-9)
                return jnp.logical_or(acc, iou > _NMS_THRESH)

            acc = iou_vreg(0, csv <= _SCORE_THRESH)
            for j in range(1, _KCAP // 16):
                acc = lax.cond(j * 16 < nk2,
                               lambda a, jj=j: iou_vreg(jj, a),
                               lambda a: a, acc)
            keep = jnp.logical_not(jnp.any(acc))
            keepv = jnp.broadcast_to(keep, (16,))

            v01 = jnp.where(iota == 0, cx1v, cy1v)
            v012 = jnp.where(iota <= 1, v01, cx2v)
            coords = jnp.where(iota <= 2, v012, cy2v)

            kvec = jnp.where(iota <= 3, coords,
                             jnp.where(iota == 4, cav, csv))
            kslot = jnp.minimum(nk2, _KCAP - 1)
            plsc.store_scatter(kbuf, [iota * _KCAP + kslot], kvec,
                               mask=jnp.logical_and(lane_lt6, keepv))

            rvec = jnp.where(iota <= 3, coords, 0.0)
            rslot = jnp.minimum(nr2, _MAX_DET)
            plsc.store_scatter(kbuf, [_REJ + iota * 128 + rslot], rvec,
                               mask=jnp.logical_and(lane_lt5,
                                                    jnp.logical_not(keepv)))
            ki = keep.astype(jnp.int32)
            return nk2 + ki, nr2 + (1 - ki)

        return lax.fori_loop(0, 16, lane_body, (nk, nr))

    nk, _ = lax.fori_loop(0, _NB, block_step, (jnp.int32(0), jnp.int32(0)))

    @pl.when(wid == 0)
    def _():
        def out_step(t, carry):
            flat = t * 16 + iota
            p = flat // 5
            c = flat - p * 5
            ck = jnp.where(c == 4, 5, c)
            src_k = ck * _KCAP + p
            src_r = _REJ + c * 128 + jnp.maximum(p - nk, 0)
            src = jnp.where(p < nk, src_k, src_r)
            obuf[pl.ds(t * 16, 16)] = plsc.load_gather(kbuf, [src])
            return carry

        lax.fori_loop(0, _OUT_FLAT // 16, out_step, jnp.int32(0))
        pltpu.sync_copy(obuf, outh)


@jax.jit
def kernel(boxes, scores):
    key = jnp.where(scores > _SCORE_THRESH, -scores, 1.0)
    order = jnp.argsort(key).astype(jnp.int32)
    bflat = boxes.reshape(-1)

    mesh = plsc.VectorSubcoreMesh(core_axis_name="c", subcore_axis_name="s",
                                  num_cores=1)
    f = pl.kernel(
        _nms_body,
        out_type=jax.ShapeDtypeStruct((_OUT_FLAT,), jnp.float32),
        mesh=mesh,
        compiler_params=pltpu.CompilerParams(needs_layout_passes=False),
        scratch_types=[
            pltpu.VMEM((4 * _N,), jnp.float32),
            pltpu.VMEM((_NPAD,), jnp.float32),
            pltpu.VMEM((_NPAD,), jnp.int32),
            pltpu.VMEM((_BUF,), jnp.float32),
            pltpu.VMEM((_OUT_FLAT,), jnp.float32),
        ],
    )
    out_flat = f(bflat, scores, order)
    return out_flat[:_MAX_DET * 5].reshape(_MAX_DET, 5)


# speedup vs baseline: 1.2033x; 1.0000x over previous
import jax
import jax.numpy as jnp
from jax import lax
from jax.experimental import pallas as pl
from jax.experimental.pallas import tpu as pltpu
from jax.experimental.pallas import tpu_sc as plsc

_SCORE_THRESH = 0.05
_NMS_THRESH = 0.5
_MAX_DET = 100

_N = 5000
_NPAD = 5120
_NG = _NPAD // 16
_NB = _NG // 16
_KCAP = 112
_REJ = 6 * _KCAP
_BUF = _REJ + 5 * 128
_OUT_FLAT = 512

_GDN = lax.GatherDimensionNumbers(
    offset_dims=(), collapsed_slice_dims=(0,), start_index_map=(0,))


def _dyn_bcast(vec, ivec):
    return lax.gather(vec, ivec[:, None], _GDN, slice_sizes=(1,),
                      mode=lax.GatherScatterMode.PROMISE_IN_BOUNDS)


def _nms_body(bh, sh, oh, outh, bv, sv, ov, kbuf, obuf):
    wid = lax.axis_index("s") + lax.axis_index("c")

    @pl.when(wid == 0)
    def _():
        pltpu.sync_copy(bh, bv)
        pltpu.sync_copy(sh, sv.at[pl.ds(0, _N)])
        pltpu.sync_copy(oh, ov.at[pl.ds(0, _N)])

    iota = lax.broadcasted_iota(jnp.int32, (16,), 0)
    neg16 = jnp.full((16,), -1.0, jnp.float32)
    n16 = jnp.full((16,), _N, jnp.int32)

    def pad_step(k, carry):
        off = jnp.minimum(5000 + k * 16, _NPAD - 16)
        sv[pl.ds(off, 16)] = neg16
        ov[pl.ds(off, 16)] = n16
        return carry

    lax.fori_loop(0, 8, pad_step, jnp.int32(0))
    lane_lt5 = iota < 5
    lane_lt6 = iota < 6
    zero16 = jnp.zeros((16,), jnp.float32)

    def zinit_step(q, carry):
        kbuf[pl.ds(q * 16, 16)] = zero16
        return carry

    lax.fori_loop(0, 5 * _KCAP // 16, zinit_step, jnp.int32(0))

    def block_step(b, st):
        run = jnp.logical_and(st[0] < _MAX_DET, wid == 0)
        return lax.cond(run, lambda: lax.fori_loop(
            b * 16, b * 16 + 16, group_step, st), lambda: st)

    def group_step(g, st):
        return lax.cond(st[0] < _MAX_DET, lambda: group_body(g, st),
                        lambda: st)

    def group_body(g, st):
        nk, nr = st
        idxv = ov[pl.ds(g * 16, 16)]
        bidx = jnp.minimum(idxv, _N - 1) * 4
        gx1 = plsc.load_gather(bv, [bidx])
        gy1 = plsc.load_gather(bv, [bidx + 1])
        gx2 = plsc.load_gather(bv, [bidx + 2])
        gy2 = plsc.load_gather(bv, [bidx + 3])
        gs = plsc.load_gather(sv, [idxv])

        def lane_body(i, st2):
            nk2, nr2 = st2
            ivec = jnp.broadcast_to(i, (16,))
            cx1v = _dyn_bcast(gx1, ivec)
            cy1v = _dyn_bcast(gy1, ivec)
            cx2v = _dyn_bcast(gx2, ivec)
            cy2v = _dyn_bcast(gy2, ivec)
            csv = _dyn_bcast(gs, ivec)
            cav = (cx2v - cx1v) * (cy2v - cy1v)

            def iou_vreg(j, acc):
                kx1j = kbuf[pl.ds(j * 16, 16)]
                ky1j = kbuf[pl.ds(_KCAP + j * 16, 16)]
                kx2j = kbuf[pl.ds(2 * _KCAP + j * 16, 16)]
                ky2j = kbuf[pl.ds(3 * _KCAP + j * 16, 16)]
                karj = kbuf[pl.ds(4 * _KCAP + j * 16, 16)]
                ltx = jnp.maximum(kx1j, cx1v)
                lty = jnp.maximum(ky1j, cy1v)
                rbx = jnp.minimum(kx2j, cx2v)
                rby = jnp.minimum(ky2j, cy2v)
                w = jnp.maximum(rbx - ltx, 0.0)
                h = jnp.maximum(rby - lty, 0.0)
                inter = w * h
                union = (karj + cav) - inter
                iou = inter / jnp.maximum(union, 1e-9)
                return jnp.logical_or(acc, iou > _NMS_THRESH)

            acc = iou_vreg(0, csv <= _SCORE_THRESH)
            for j in range(1, _KCAP // 16):
                acc = lax.cond(j * 16 < nk2,
                               lambda a, jj=j: iou_vreg(jj, a),
                               lambda a: a, acc)
            keep = jnp.logical_not(jnp.any(acc))
            keepv = jnp.broadcast_to(keep, (16,))

            v01 = jnp.where(iota == 0, cx1v, cy1v)
            v012 = jnp.where(iota <= 1, v01, cx2v)
            coords = jnp.where(iota <= 2, v012, cy2v)

            kvec = jnp.where(iota <= 3, coords,
                             jnp.where(iota == 4, cav, csv))
            kslot = jnp.minimum(nk2, _KCAP - 1)
            plsc.store_scatter(kbuf, [iota * _KCAP + kslot], kvec,
                               mask=jnp.logical_and(lane_lt6, keepv))

            rvec = jnp.where(iota <= 3, coords, 0.0)
            rslot = jnp.minimum(nr2, _MAX_DET)
            plsc.store_scatter(kbuf, [_REJ + iota * 128 + rslot], rvec,
                               mask=jnp.logical_and(lane_lt5,
                                                    jnp.logical_not(keepv)))
            ki = keep.astype(jnp.int32)
            return nk2 + ki, nr2 + (1 - ki)

        return lax.fori_loop(0, 16, lane_body, (nk, nr))

    nk, _ = lax.fori_loop(0, _NB, block_step, (jnp.int32(0), jnp.int32(0)))

    @pl.when(wid == 0)
    def _():
        def out_step(t, carry):
            flat = t * 16 + iota
            p = flat // 5
            c = flat - p * 5
            ck = jnp.where(c == 4, 5, c)
            src_k = ck * _KCAP + p
            src_r = _REJ + c * 128 + jnp.maximum(p - nk, 0)
            src = jnp.where(p < nk, src_k, src_r)
            obuf[pl.ds(t * 16, 16)] = plsc.load_gather(kbuf, [src])
            return carry

        lax.fori_loop(0, _OUT_FLAT // 16, out_step, jnp.int32(0))
        pltpu.sync_copy(obuf, outh)


@jax.jit
def kernel(boxes, scores):
    key = jnp.where(scores > _SCORE_THRESH, -scores, 1.0)
    order = jnp.argsort(key).astype(jnp.int32)
    bflat = boxes.reshape(-1)

    mesh = plsc.VectorSubcoreMesh(core_axis_name="c", subcore_axis_name="s",
                                  num_cores=1, num_subcores=1)
    f = pl.kernel(
        _nms_body,
        out_type=jax.ShapeDtypeStruct((_OUT_FLAT,), jnp.float32),
        mesh=mesh,
        compiler_params=pltpu.CompilerParams(needs_layout_passes=False),
        scratch_types=[
            pltpu.VMEM((4 * _N,), jnp.float32),
            pltpu.VMEM((_NPAD,), jnp.float32),
            pltpu.VMEM((_NPAD,), jnp.int32),
            pltpu.VMEM((_BUF,), jnp.float32),
            pltpu.VMEM((_OUT_FLAT,), jnp.float32),
        ],
    )
    out_flat = f(bflat, scores, order)
    return out_flat[:_MAX_DET * 5].reshape(_MAX_DET, 5)
